# Initial kernel scaffold; baseline (speedup 1.0000x reference)
import jax
import jax.numpy as jnp
from jax.experimental import pallas as pl


def kernel(tet_v, sdf, deform, tet_ind):
    raise NotImplementedError("write your pallas kernel here")


# speedup vs baseline: 19.6256x; 19.6256x over previous
import jax
import jax.numpy as jnp
import numpy as np
from jax import lax
from jax.experimental import pallas as pl
from jax.experimental.pallas import tpu as pltpu
from jax.experimental.pallas import tpu_sc as plsc

GRID_SCALE = 0.0001

TRI_TABLE = np.array([
    [-1, -1, -1, -1, -1, -1],
    [1, 0, 2, -1, -1, -1],
    [4, 0, 3, -1, -1, -1],
    [1, 4, 2, 1, 3, 4],
    [3, 1, 5, -1, -1, -1],
    [2, 3, 0, 2, 5, 3],
    [1, 4, 0, 1, 5, 4],
    [4, 2, 5, -1, -1, -1],
    [4, 5, 2, -1, -1, -1],
    [4, 1, 0, 4, 5, 1],
    [3, 2, 0, 3, 5, 2],
    [1, 3, 5, -1, -1, -1],
    [4, 1, 2, 4, 3, 1],
    [3, 0, 4, -1, -1, -1],
    [2, 0, 1, -1, -1, -1],
    [-1, -1, -1, -1, -1, -1]], dtype=np.int32)

EDGE_A = (0, 0, 0, 1, 1, 2)
EDGE_B = (1, 2, 3, 2, 3, 3)

NC = 2
NS = 16
L = 16
NW = NC * NS

F = 300000
N = 100000

C = 640
G = C // L
CH = 15
TPW = C * CH
F_PAD = TPW * NW


def _pack_body(tv_ref, df_ref, ox_ref, oy_ref, oz_ref):
    v = tv_ref[...] + jnp.tanh(df_ref[...]) * (GRID_SCALE / 2.0)
    ox_ref[...] = v[:, 0:1]
    oy_ref[...] = v[:, 1:2]
    oz_ref[...] = v[:, 2:3]


def _deform_verts(tet_v, deform):
    bn = 2000
    shp = jax.ShapeDtypeStruct((N, 1), jnp.float32)
    spec = pl.BlockSpec((bn, 1), lambda i: (i, 0))
    ox, oy, oz = pl.pallas_call(
        _pack_body,
        grid=(N // bn,),
        in_specs=[
            pl.BlockSpec((bn, 3), lambda i: (i, 0)),
            pl.BlockSpec((bn, 3), lambda i: (i, 0)),
        ],
        out_specs=[spec, spec, spec],
        out_shape=[shp, shp, shp],
    )(tet_v, deform)
    return ox.reshape(N), oy.reshape(N), oz.reshape(N)


def _mt_body(x_hbm, y_hbm, z_hbm, s_hbm, ti_hbm, tt_hbm,
             mv_hbm, fc_hbm, vl_hbm,
             i0_v, i1_v, i2_v, i3_v,
             gx0, gx1, gx2, gx3, gy0, gy1, gy2, gy3,
             gz0, gz1, gz2, gz3, gs0, gs1, gs2, gs3,
             tt_v, mv_v, fc_v, vl_v, sem):
    wid = lax.axis_index("s") * NC + lax.axis_index("c")
    wbase = wid * TPW

    idx_bufs = (i0_v, i1_v, i2_v, i3_v)
    gx = (gx0, gx1, gx2, gx3)
    gy = (gy0, gy1, gy2, gy3)
    gz = (gz0, gz1, gz2, gz3)
    gs = (gs0, gs1, gs2, gs3)

    pltpu.sync_copy(tt_hbm, tt_v)

    lane = lax.broadcasted_iota(jnp.int32, (L,), 0)

    def splat_i(k):
        return jnp.full((L,), k, jnp.int32)

    def chunk_body(g, carry):
        t0 = wbase + g * C
        for a in range(4):
            pltpu.sync_copy(ti_hbm.at[a, pl.ds(t0, C)], idx_bufs[a])
        copies = []
        for a in range(4):
            copies.append(pltpu.async_copy(x_hbm.at[idx_bufs[a]], gx[a], sem))
            copies.append(pltpu.async_copy(y_hbm.at[idx_bufs[a]], gy[a], sem))
            copies.append(pltpu.async_copy(z_hbm.at[idx_bufs[a]], gz[a], sem))
            copies.append(pltpu.async_copy(s_hbm.at[idx_bufs[a]], gs[a], sem))
        for cp in copies:
            cp.wait()

        def group_body(j, c2):
            o = pl.ds(j * L, L)
            tloc = j * L + lane
            X = [gx[a][o] for a in range(4)]
            Y = [gy[a][o] for a in range(4)]
            Z = [gz[a][o] for a in range(4)]
            S = [gs[a][o] for a in range(4)]

            code = (S[0] > 0.0).astype(jnp.int32)
            code = code + 2 * (S[1] > 0.0).astype(jnp.int32)
            code = code + 4 * (S[2] > 0.0).astype(jnp.int32)
            code = code + 8 * (S[3] > 0.0).astype(jnp.int32)
            code6 = code * 6

            gt6 = (t0 + tloc) * 6
            f6 = tloc * 6
            valid = []
            for col in range(6):
                ent = plsc.load_gather(tt_v, [code6 + col])
                v_ok = ent >= 0
                valid.append(v_ok)
                fval = gt6 + jnp.where(v_ok, ent, 0)
                plsc.store_scatter(fc_v, [f6 + col], fval)
            for r in range(2):
                v_ok = valid[3 * r] & valid[3 * r + 1] & valid[3 * r + 2]
                plsc.store_scatter(vl_v, [2 * tloc + r],
                                   v_ok.astype(jnp.int32))

            m18 = tloc * 18
            for e in range(6):
                a, b = EDGE_A[e], EDGE_B[e]
                d = S[b] - S[a]
                small = jnp.abs(d) < 1e-10
                w = jnp.where(small, 0.5,
                              S[b] / jnp.where(small, 1.0, d))
                u = 1.0 - w
                base = m18 + e * 3
                plsc.store_scatter(mv_v, [base], X[a] * w + X[b] * u)
                plsc.store_scatter(mv_v, [base + 1], Y[a] * w + Y[b] * u)
                plsc.store_scatter(mv_v, [base + 2], Z[a] * w + Z[b] * u)
            return c2

        lax.fori_loop(0, G, group_body, 0)

        pltpu.sync_copy(mv_v, mv_hbm.at[pl.ds(t0 * 18, 18 * C)])
        pltpu.sync_copy(fc_v, fc_hbm.at[pl.ds(t0 * 6, 6 * C)])
        pltpu.sync_copy(vl_v, vl_hbm.at[pl.ds(t0 * 2, 2 * C)])
        return carry

    lax.fori_loop(0, CH, chunk_body, 0)


def _marching(x, y, z, s, ti_t, tt):
    mesh = plsc.VectorSubcoreMesh(core_axis_name="c", subcore_axis_name="s")
    idx_t = pltpu.VMEM((C,), jnp.int32)
    val_t = pltpu.VMEM((C,), jnp.float32)
    fn = pl.kernel(
        _mt_body, mesh=mesh,
        out_type=[
            jax.ShapeDtypeStruct((F_PAD * 18,), jnp.float32),
            jax.ShapeDtypeStruct((F_PAD * 6,), jnp.int32),
            jax.ShapeDtypeStruct((F_PAD * 2,), jnp.int32),
        ],
        scratch_types=[
            idx_t, idx_t, idx_t, idx_t,
            val_t, val_t, val_t, val_t,
            val_t, val_t, val_t, val_t,
            val_t, val_t, val_t, val_t,
            val_t, val_t, val_t, val_t,
            pltpu.VMEM((96,), jnp.int32),
            pltpu.VMEM((18 * C,), jnp.float32),
            pltpu.VMEM((6 * C,), jnp.int32),
            pltpu.VMEM((2 * C,), jnp.int32),
            pltpu.SemaphoreType.DMA,
        ],
        compiler_params=pltpu.CompilerParams(needs_layout_passes=False),
    )
    return fn(x, y, z, s, ti_t, tt)


def kernel(tet_v, sdf, deform, tet_ind):
    x, y, z = _deform_verts(tet_v, deform)
    ti_t = jnp.pad(tet_ind, ((0, F_PAD - F), (0, 0))).T
    tt = jnp.asarray(TRI_TABLE).reshape(96)
    mv, fc, vl = _marching(x, y, z, sdf, ti_t, tt)
    mesh_verts = mv.reshape(F_PAD * 6, 3)[:F * 6]
    faces = fc.reshape(F_PAD * 2, 3)[:F * 2]
    face_valid = vl[:F * 2].astype(bool)
    return mesh_verts, faces, face_valid
